# baseline (speedup 1.0000x reference)
import jax
import jax.numpy as jnp
from jax.experimental import pallas as pl

_BN = 2000


def _fused_kernel(x_ref, wt_ref, b_ref, o_ref):
    acc = jnp.dot(x_ref[...], wt_ref[...], preferred_element_type=jnp.float32)
    o_ref[...] = jnp.maximum(acc + b_ref[...], 0.0)


def kernel(x, W, b):
    n, d_in = x.shape
    d_hid = W.shape[0]
    wt = W.T
    b2 = b.reshape(1, d_hid)
    grid = (n // _BN,)
    return pl.pallas_call(
        _fused_kernel,
        grid=grid,
        in_specs=[
            pl.BlockSpec((_BN, d_in), lambda i: (i, 0)),
            pl.BlockSpec((d_in, d_hid), lambda i: (0, 0)),
            pl.BlockSpec((1, d_hid), lambda i: (0, 0)),
        ],
        out_specs=pl.BlockSpec((_BN, d_hid), lambda i: (i, 0)),
        out_shape=jax.ShapeDtypeStruct((n, d_hid), x.dtype),
    )(x, wt, b2)


# speedup vs baseline: 1.0019x; 1.0019x over previous
import jax
import jax.numpy as jnp
from jax.experimental import pallas as pl
from jax.experimental.pallas import tpu as pltpu

_BN = 2000


def _fused_kernel(x_ref, wt_ref, b_ref, o_ref):
    acc = jnp.dot(x_ref[...], wt_ref[...], preferred_element_type=jnp.float32)
    o_ref[...] = jnp.maximum(acc + b_ref[...], 0.0)


def kernel(x, W, b):
    n, d_in = x.shape
    d_hid = W.shape[0]
    wt = W.T
    b2 = b.reshape(1, d_hid)
    grid = (n // _BN,)
    return pl.pallas_call(
        _fused_kernel,
        grid=grid,
        in_specs=[
            pl.BlockSpec((_BN, d_in), lambda i: (i, 0)),
            pl.BlockSpec((d_in, d_hid), lambda i: (0, 0)),
            pl.BlockSpec((1, d_hid), lambda i: (0, 0)),
        ],
        out_specs=pl.BlockSpec((_BN, d_hid), lambda i: (i, 0)),
        out_shape=jax.ShapeDtypeStruct((n, d_hid), x.dtype),
        compiler_params=pltpu.CompilerParams(
            dimension_semantics=("parallel",),
        ),
    )(x, wt, b2)


# speedup vs baseline: 1.4007x; 1.3980x over previous
import jax
import jax.numpy as jnp
from jax.experimental import pallas as pl
from jax.experimental.pallas import tpu as pltpu

_BN = 5000


def _fused_kernel(x_ref, wt_ref, b_ref, o_ref):
    acc = jnp.dot(x_ref[...], wt_ref[...], preferred_element_type=jnp.float32)
    o_ref[...] = jnp.maximum(acc + b_ref[...], 0.0)


def kernel(x, W, b):
    n, d_in = x.shape
    d_hid = W.shape[0]
    wt = W.T
    b2 = b.reshape(1, d_hid)
    grid = (n // _BN,)
    return pl.pallas_call(
        _fused_kernel,
        grid=grid,
        in_specs=[
            pl.BlockSpec((_BN, d_in), lambda i: (i, 0)),
            pl.BlockSpec((d_in, d_hid), lambda i: (0, 0)),
            pl.BlockSpec((1, d_hid), lambda i: (0, 0)),
        ],
        out_specs=pl.BlockSpec((_BN, d_hid), lambda i: (i, 0)),
        out_shape=jax.ShapeDtypeStruct((n, d_hid), x.dtype),
        compiler_params=pltpu.CompilerParams(
            dimension_semantics=("parallel",),
        ),
    )(x, wt, b2)


# speedup vs baseline: 1.6190x; 1.1559x over previous
import jax
import jax.numpy as jnp
from jax.experimental import pallas as pl
from jax.experimental.pallas import tpu as pltpu

_BN = 10000


def _fused_kernel(x_ref, wt_ref, b_ref, o_ref):
    acc = jnp.dot(x_ref[...], wt_ref[...], preferred_element_type=jnp.float32)
    o_ref[...] = jnp.maximum(acc + b_ref[...], 0.0)


def kernel(x, W, b):
    n, d_in = x.shape
    d_hid = W.shape[0]
    wt = W.T
    b2 = b.reshape(1, d_hid)
    grid = (n // _BN,)
    return pl.pallas_call(
        _fused_kernel,
        grid=grid,
        in_specs=[
            pl.BlockSpec((_BN, d_in), lambda i: (i, 0)),
            pl.BlockSpec((d_in, d_hid), lambda i: (0, 0)),
            pl.BlockSpec((1, d_hid), lambda i: (0, 0)),
        ],
        out_specs=pl.BlockSpec((_BN, d_hid), lambda i: (i, 0)),
        out_shape=jax.ShapeDtypeStruct((n, d_hid), x.dtype),
        compiler_params=pltpu.CompilerParams(
            dimension_semantics=("parallel",),
        ),
    )(x, wt, b2)


# speedup vs baseline: 1.6889x; 1.0431x over previous
import jax
import jax.numpy as jnp
from jax.experimental import pallas as pl
from jax.experimental.pallas import tpu as pltpu

_BN = 20000


def _fused_kernel(x_ref, wt_ref, b_ref, o_ref):
    acc = jnp.dot(x_ref[...], wt_ref[...], preferred_element_type=jnp.float32)
    o_ref[...] = jnp.maximum(acc + b_ref[...], 0.0)


def kernel(x, W, b):
    n, d_in = x.shape
    d_hid = W.shape[0]
    wt = W.T
    b2 = b.reshape(1, d_hid)
    grid = (n // _BN,)
    return pl.pallas_call(
        _fused_kernel,
        grid=grid,
        in_specs=[
            pl.BlockSpec((_BN, d_in), lambda i: (i, 0)),
            pl.BlockSpec((d_in, d_hid), lambda i: (0, 0)),
            pl.BlockSpec((1, d_hid), lambda i: (0, 0)),
        ],
        out_specs=pl.BlockSpec((_BN, d_hid), lambda i: (i, 0)),
        out_shape=jax.ShapeDtypeStruct((n, d_hid), x.dtype),
        compiler_params=pltpu.CompilerParams(
            dimension_semantics=("parallel",),
        ),
    )(x, wt, b2)
